# baseline (speedup 1.0000x reference)
import functools

import jax
import jax.numpy as jnp
import numpy as np
from jax import lax
from jax.experimental import pallas as pl
from jax.experimental.pallas import tpu as pltpu
from jax.experimental.pallas import tpu_sc as plsc

N = 128
SLAB = 64
NSLAB = N // SLAB
BIG = 1 << 17

NWORK = 32
NVOX = N * N * N
VPW = NVOX // NWORK
CHUNK = 16384
NCHUNK = VPW // CHUNK
NBINS = 320

_THING = np.zeros((1, 16), np.int32)
for _c in (1, 2, 3, 4, 5, 6, 7, 8, 9, 12, 13):
    _THING[0, _c] = 1


def _hist_body(inst_hbm, sem_hbm, out_hbm,
               inst_buf, sem_buf, hist_buf, acc_buf):
    cid = lax.axis_index("c")
    sid = lax.axis_index("s")
    wid = sid * 2 + cid

    def _zero(i, _):
        hist_buf[pl.ds(i * 16, 16)] = jnp.zeros((16,), jnp.int32)
        return 0
    lax.fori_loop(0, 16 * NBINS // 16, _zero, 0)

    lanes = lax.broadcasted_iota(jnp.int32, (16,), 0)
    bank = lanes * NBINS
    ones = jnp.ones((16,), jnp.int32)
    base = wid * VPW

    def _chunk(c, _):
        off = base + c * CHUNK
        pltpu.sync_copy(inst_hbm.at[pl.ds(off, CHUNK)], inst_buf)
        pltpu.sync_copy(sem_hbm.at[pl.ds(off, CHUNK)], sem_buf)

        def _vec(i, __):
            b = i * 128
            for u in range(8):
                iv = inst_buf[pl.ds(b + u * 16, 16)]
                sv = sem_buf[pl.ds(b + u * 16, 16)]
                plsc.addupdate_scatter(
                    hist_buf, [bank + iv * 16 + sv], ones)
            return 0
        lax.fori_loop(0, CHUNK // 128, _vec, 0)
        return 0
    lax.fori_loop(0, NCHUNK, _chunk, 0)

    for f in range(20):
        acc = hist_buf[pl.ds(0 * NBINS + f * 16, 16)]
        for l in range(1, 16):
            acc = acc + hist_buf[pl.ds(l * NBINS + f * 16, 16)]
        acc_buf[f, :] = acc
    pltpu.sync_copy(acc_buf, out_hbm.at[wid])


def _hist_call(inst_flat, sem_flat):
    mesh = plsc.VectorSubcoreMesh(core_axis_name="c", subcore_axis_name="s")
    fn = pl.kernel(
        _hist_body,
        mesh=mesh,
        compiler_params=pltpu.CompilerParams(needs_layout_passes=False),
        out_type=jax.ShapeDtypeStruct((NWORK, 20, 16), jnp.int32),
        scratch_types=[
            pltpu.VMEM((CHUNK,), jnp.int32),
            pltpu.VMEM((CHUNK,), jnp.int32),
            pltpu.VMEM((16 * NBINS,), jnp.int32),
            pltpu.VMEM((20, 16), jnp.int32),
        ],
    )
    return fn(inst_flat, sem_flat)


def _maps_body(ids_ref, hist_ref, maps_ref):
    m2 = jnp.int32(0)
    for k in range(12):
        m2 = m2 | (jnp.int32(1) << (ids_ref[k] + jnp.int32(1)))
    raw = jnp.sum(hist_ref[...], axis=0)
    rowio20 = lax.broadcasted_iota(jnp.int32, (20, 1), 0)
    member = ((m2 >> rowio20) & 1) == 1
    hsm = jnp.where(member, raw, 0)
    row0 = jnp.broadcast_to(NVOX - jnp.sum(hsm), (1, 16)).astype(jnp.int32)
    hs = jnp.concatenate([row0, hsm[1:20], jnp.zeros((1, 16), jnp.int32)],
                         axis=0)
    rowtot = jnp.sum(hs, axis=1, keepdims=True)
    present = rowtot > 0
    rowio = lax.broadcasted_iota(jnp.int32, (21, 1), 0)
    presf = jnp.where(present & (rowio >= 1), 1, 0)
    rank0 = jnp.where(present[0:1, 0:1], 1, 0)

    cums = presf
    for d in (1, 2, 4, 8, 16):
        cums = cums + jnp.concatenate(
            [jnp.zeros((d, 1), jnp.int32), cums[: 21 - d]], axis=0)
    prior = cums - presf
    pid = 2 + rank0 + prior

    lane1 = lax.broadcasted_iota(jnp.int32, (1, 16), 1)
    maskt_b = ((lane1 >= 1) & (lane1 <= 9)) | (lane1 == 12) | (lane1 == 13)
    maskt = maskt_b.astype(jnp.int32)
    masked = hs * maskt
    has = jnp.sum(masked, axis=1, keepdims=True) > 0
    mx = jnp.max(masked, axis=1, keepdims=True)
    laneio = lax.broadcasted_iota(jnp.int32, (21, 16), 1)
    cand = jnp.where((masked == mx) & maskt_b, laneio, 999)
    mode = jnp.min(cand, axis=1, keepdims=True)
    semv = jnp.where(pid == 2, 11, jnp.where(has, mode, 0))
    packed = pid + semv * 32

    padded = jnp.concatenate(
        [jnp.zeros((2, 1), jnp.int32), packed, jnp.zeros((9, 1), jnp.int32)],
        axis=0)
    rowio32 = lax.broadcasted_iota(jnp.int32, (32, 1), 0)
    lut = jnp.where(
        rowio32 == 1, 1 + 10 * 32,
        jnp.where(rowio32 == 2, 2 + 11 * 32,
                  jnp.where(rowio32 == 0, 0, padded)))
    maps_ref[...] = jnp.concatenate(
        [lut[3 * j:3 * j + 1] + 512 * lut[3 * j + 1:3 * j + 2]
         + 262144 * lut[3 * j + 2:3 * j + 3] for j in range(8)], axis=0)


def _maps_call(ids_pad, hist):
    return pl.pallas_call(
        _maps_body,
        in_specs=[pl.BlockSpec(memory_space=pltpu.SMEM),
                  pl.BlockSpec((NWORK, 20, 16), lambda: (0, 0, 0))],
        out_shape=jax.ShapeDtypeStruct((8, 1), jnp.int32),
    )(ids_pad, hist)


def _shift(a, s, axis):
    if s == 0:
        return a
    n = a.shape[axis]
    if s > 0:
        main = lax.slice_in_dim(a, s, n, axis=axis)
        edge = lax.slice_in_dim(a, n - 1, n, axis=axis)
        return jnp.concatenate([main] + [edge] * s, axis=axis)
    main = lax.slice_in_dim(a, 0, n + s, axis=axis)
    edge = lax.slice_in_dim(a, 0, 1, axis=axis)
    return jnp.concatenate([edge] * (-s) + [main], axis=axis)


SENT = 65536


def _stencil_body(ids_ref, g_ref, inst_ref, sem_ref, pk_ref):
    m2 = jnp.int32(0)
    for k in range(12):
        m2 = m2 | (jnp.int32(1) << (ids_ref[k] + jnp.int32(1)))
    inst = inst_ref[...]
    sem = sem_ref[...]
    member = ((m2 >> inst) & 1) == 1
    canon = jnp.where(sem == 10, 1,
                      jnp.where(sem == 11, 2,
                                jnp.where(member, inst + 2, 0)))
    surface = jnp.abs(g_ref[...]) <= 1.5

    p = jnp.where(canon > 0, canon, BIG)
    acc = None
    for zi in range(6):
        term = _shift(p, zi - 3, 2) + 256 * zi
        acc = term if acc is None else jnp.minimum(acc, term)
    t1 = acc
    acc = None
    for yi in range(6):
        term = _shift(t1, yi - 3, 1) + 1536 * yi
        acc = term if acc is None else jnp.minimum(acc, term)
    pk_ref[...] = ((jnp.minimum(acc, SENT) << 6) | (canon << 1)
                   | surface.astype(jnp.int32))


def _stencil_call(ids_pad, geometry, instances, semantics):
    bs = lambda: pl.BlockSpec((SLAB, N, N), lambda s: (s, 0, 0))
    return pl.pallas_call(
        _stencil_body,
        grid=(NSLAB,),
        in_specs=[pl.BlockSpec(memory_space=pltpu.SMEM), bs(), bs(), bs()],
        out_specs=bs(),
        out_shape=jax.ShapeDtypeStruct((N, N, N), jnp.int32),
    )(ids_pad, geometry, instances, semantics)


def _final_body(maps_ref, prev_ref, cur_ref, next_ref, oi_ref, os_ref):
    s = pl.program_id(0)
    cur = cur_ref[...]
    ext = jnp.concatenate([prev_ref[SLAB - 3:SLAB], cur, next_ref[0:2]],
                          axis=0)
    rowio = lax.broadcasted_iota(jnp.int32, (SLAB + 5, N, N), 0)
    gi = s * SLAB - 3 + rowio
    ext = jnp.where(gi < 0, cur_ref[0:1], ext)
    ext = jnp.where(gi > N - 1, cur_ref[SLAB - 1:SLAB], ext)

    acc = None
    for xi in range(6):
        term = (lax.slice_in_dim(ext, xi, xi + SLAB, axis=0) >> 6) + 9216 * xi
        acc = term if acc is None else jnp.minimum(acc, term)

    bits = cur & 63
    canon = bits >> 1
    label = jnp.where(acc < SENT, acc & 255, 0)
    cf = jnp.where(bits == 1, label, canon)

    j = (cf * 21846) >> 16
    r = cf - j * 3
    b0 = (j & 1) == 1
    b1 = (j & 2) == 2
    b2 = (j & 4) == 4
    m01 = jnp.where(b0, maps_ref[1, 0], maps_ref[0, 0])
    m23 = jnp.where(b0, maps_ref[3, 0], maps_ref[2, 0])
    m45 = jnp.where(b0, maps_ref[5, 0], maps_ref[4, 0])
    m67 = jnp.where(b0, maps_ref[7, 0], maps_ref[6, 0])
    m03 = jnp.where(b1, m23, m01)
    m47 = jnp.where(b1, m67, m45)
    w = jnp.where(b2, m47, m03)
    v = (w >> (r * 9)) & 511
    oi_ref[...] = v & 31
    os_ref[...] = v >> 5


def _final_call(maps, packed):
    bs = lambda im: pl.BlockSpec((SLAB, N, N), im)
    return pl.pallas_call(
        _final_body,
        grid=(NSLAB,),
        in_specs=[
            pl.BlockSpec(memory_space=pltpu.SMEM),
            bs(lambda s: (jnp.maximum(s - 1, 0), 0, 0)),
            bs(lambda s: (s, 0, 0)),
            bs(lambda s: (jnp.minimum(s + 1, NSLAB - 1), 0, 0)),
        ],
        out_specs=[bs(lambda s: (s, 0, 0)), bs(lambda s: (s, 0, 0))],
        out_shape=[jax.ShapeDtypeStruct((N, N, N), jnp.int32)] * 2,
    )(maps, packed, packed, packed)


def kernel(geometry, instances, semantics, instance_ids_2d):
    ids_pad = jnp.concatenate(
        [instance_ids_2d.astype(jnp.int32), jnp.full((4,), 30, jnp.int32)])
    packed = _stencil_call(ids_pad, geometry, instances, semantics)
    hist = _hist_call(instances.reshape(-1), semantics.reshape(-1))
    maps = _maps_call(ids_pad, hist)
    out_inst, out_sem = _final_call(maps, packed)
    return out_inst, out_sem


# speedup vs baseline: 1.0546x; 1.0546x over previous
import functools

import jax
import jax.numpy as jnp
import numpy as np
from jax import lax
from jax.experimental import pallas as pl
from jax.experimental.pallas import tpu as pltpu
from jax.experimental.pallas import tpu_sc as plsc

N = 128
SLAB = 32
NSLAB = N // SLAB
BIG = 1 << 17

NWORK = 32
NVOX = N * N * N
VPW = NVOX // NWORK
CHUNK = 32768
NCHUNK = VPW // CHUNK
NBINS = 320

_THING = np.zeros((1, 16), np.int32)
for _c in (1, 2, 3, 4, 5, 6, 7, 8, 9, 12, 13):
    _THING[0, _c] = 1


def _hist_body(inst_hbm, sem_hbm, out_hbm,
               inst_buf, sem_buf, hist_buf, acc_buf):
    cid = lax.axis_index("c")
    sid = lax.axis_index("s")
    wid = sid * 2 + cid

    def _zero(i, _):
        hist_buf[pl.ds(i * 16, 16)] = jnp.zeros((16,), jnp.int32)
        return 0
    lax.fori_loop(0, 16 * NBINS // 16, _zero, 0)

    lanes = lax.broadcasted_iota(jnp.int32, (16,), 0)
    bank = lanes * NBINS
    ones = jnp.ones((16,), jnp.int32)
    base = wid * VPW

    def _chunk(c, _):
        off = base + c * CHUNK
        pltpu.sync_copy(inst_hbm.at[pl.ds(off, CHUNK)], inst_buf)
        pltpu.sync_copy(sem_hbm.at[pl.ds(off, CHUNK)], sem_buf)

        def _vec(i, __):
            b = i * 128
            for u in range(8):
                iv = inst_buf[pl.ds(b + u * 16, 16)]
                sv = sem_buf[pl.ds(b + u * 16, 16)]
                plsc.addupdate_scatter(
                    hist_buf, [bank + iv * 16 + sv], ones)
            return 0
        lax.fori_loop(0, CHUNK // 128, _vec, 0)
        return 0
    lax.fori_loop(0, NCHUNK, _chunk, 0)

    for f in range(20):
        acc = hist_buf[pl.ds(0 * NBINS + f * 16, 16)]
        for l in range(1, 16):
            acc = acc + hist_buf[pl.ds(l * NBINS + f * 16, 16)]
        acc_buf[f, :] = acc
    pltpu.sync_copy(acc_buf, out_hbm.at[wid])


def _hist_call(inst_flat, sem_flat):
    mesh = plsc.VectorSubcoreMesh(core_axis_name="c", subcore_axis_name="s")
    fn = pl.kernel(
        _hist_body,
        mesh=mesh,
        compiler_params=pltpu.CompilerParams(needs_layout_passes=False),
        out_type=jax.ShapeDtypeStruct((NWORK, 20, 16), jnp.int32),
        scratch_types=[
            pltpu.VMEM((CHUNK,), jnp.int32),
            pltpu.VMEM((CHUNK,), jnp.int32),
            pltpu.VMEM((16 * NBINS,), jnp.int32),
            pltpu.VMEM((20, 16), jnp.int32),
        ],
    )
    return fn(inst_flat, sem_flat)


def _maps_body(ids_ref, hist_ref, maps_ref):
    m2 = jnp.int32(0)
    for k in range(12):
        m2 = m2 | (jnp.int32(1) << (ids_ref[k] + jnp.int32(1)))
    raw = jnp.sum(hist_ref[...], axis=0)
    rowio20 = lax.broadcasted_iota(jnp.int32, (20, 1), 0)
    member = ((m2 >> rowio20) & 1) == 1
    hsm = jnp.where(member, raw, 0)
    row0 = jnp.broadcast_to(NVOX - jnp.sum(hsm), (1, 16)).astype(jnp.int32)
    hs = jnp.concatenate([row0, hsm[1:20], jnp.zeros((1, 16), jnp.int32)],
                         axis=0)
    rowtot = jnp.sum(hs, axis=1, keepdims=True)
    present = rowtot > 0
    rowio = lax.broadcasted_iota(jnp.int32, (21, 1), 0)
    presf = jnp.where(present & (rowio >= 1), 1, 0)
    rank0 = jnp.where(present[0:1, 0:1], 1, 0)

    cums = presf
    for d in (1, 2, 4, 8, 16):
        cums = cums + jnp.concatenate(
            [jnp.zeros((d, 1), jnp.int32), cums[: 21 - d]], axis=0)
    prior = cums - presf
    pid = 2 + rank0 + prior

    lane1 = lax.broadcasted_iota(jnp.int32, (1, 16), 1)
    maskt_b = ((lane1 >= 1) & (lane1 <= 9)) | (lane1 == 12) | (lane1 == 13)
    maskt = maskt_b.astype(jnp.int32)
    masked = hs * maskt
    has = jnp.sum(masked, axis=1, keepdims=True) > 0
    mx = jnp.max(masked, axis=1, keepdims=True)
    laneio = lax.broadcasted_iota(jnp.int32, (21, 16), 1)
    cand = jnp.where((masked == mx) & maskt_b, laneio, 999)
    mode = jnp.min(cand, axis=1, keepdims=True)
    semv = jnp.where(pid == 2, 11, jnp.where(has, mode, 0))
    packed = pid + semv * 32

    padded = jnp.concatenate(
        [jnp.zeros((2, 1), jnp.int32), packed, jnp.zeros((9, 1), jnp.int32)],
        axis=0)
    rowio32 = lax.broadcasted_iota(jnp.int32, (32, 1), 0)
    lut = jnp.where(
        rowio32 == 1, 1 + 10 * 32,
        jnp.where(rowio32 == 2, 2 + 11 * 32,
                  jnp.where(rowio32 == 0, 0, padded)))
    maps_ref[...] = jnp.concatenate(
        [lut[3 * j:3 * j + 1] + 512 * lut[3 * j + 1:3 * j + 2]
         + 262144 * lut[3 * j + 2:3 * j + 3] for j in range(8)], axis=0)


def _maps_call(ids_pad, hist):
    return pl.pallas_call(
        _maps_body,
        in_specs=[pl.BlockSpec(memory_space=pltpu.SMEM),
                  pl.BlockSpec((NWORK, 20, 16), lambda: (0, 0, 0))],
        out_shape=jax.ShapeDtypeStruct((8, 1), jnp.int32),
    )(ids_pad, hist)


def _shift(a, s, axis):
    if s == 0:
        return a
    n = a.shape[axis]
    if s > 0:
        main = lax.slice_in_dim(a, s, n, axis=axis)
        edge = lax.slice_in_dim(a, n - 1, n, axis=axis)
        return jnp.concatenate([main] + [edge] * s, axis=axis)
    main = lax.slice_in_dim(a, 0, n + s, axis=axis)
    edge = lax.slice_in_dim(a, 0, 1, axis=axis)
    return jnp.concatenate([edge] * (-s) + [main], axis=axis)


SENT = 65536


def _stencil_body(ids_ref, g_ref, inst_ref, sem_ref, pk_ref):
    m2 = jnp.int32(0)
    for k in range(12):
        m2 = m2 | (jnp.int32(1) << (ids_ref[k] + jnp.int32(1)))
    inst = inst_ref[...]
    sem = sem_ref[...]
    member = ((m2 >> inst) & 1) == 1
    canon = jnp.where(sem == 10, 1,
                      jnp.where(sem == 11, 2,
                                jnp.where(member, inst + 2, 0)))
    surface = jnp.abs(g_ref[...]) <= 1.5

    p = jnp.where(canon > 0, canon, BIG)
    acc = None
    for zi in range(6):
        term = _shift(p, zi - 3, 2) + 256 * zi
        acc = term if acc is None else jnp.minimum(acc, term)
    t1 = acc
    acc = None
    for yi in range(6):
        term = _shift(t1, yi - 3, 1) + 1536 * yi
        acc = term if acc is None else jnp.minimum(acc, term)
    pk_ref[...] = ((jnp.minimum(acc, SENT) << 6) | (canon << 1)
                   | surface.astype(jnp.int32))


def _stencil_call(ids_pad, geometry, instances, semantics):
    bs = lambda: pl.BlockSpec((SLAB, N, N), lambda s: (s, 0, 0))
    return pl.pallas_call(
        _stencil_body,
        grid=(NSLAB,),
        in_specs=[pl.BlockSpec(memory_space=pltpu.SMEM), bs(), bs(), bs()],
        out_specs=bs(),
        out_shape=jax.ShapeDtypeStruct((N, N, N), jnp.int32),
    )(ids_pad, geometry, instances, semantics)


def _final_body(maps_ref, prev_ref, cur_ref, next_ref, oi_ref, os_ref):
    s = pl.program_id(0)
    cur = cur_ref[...]
    ext = jnp.concatenate([prev_ref[SLAB - 3:SLAB], cur, next_ref[0:2]],
                          axis=0)
    rowio = lax.broadcasted_iota(jnp.int32, (SLAB + 5, N, N), 0)
    gi = s * SLAB - 3 + rowio
    ext = jnp.where(gi < 0, cur_ref[0:1], ext)
    ext = jnp.where(gi > N - 1, cur_ref[SLAB - 1:SLAB], ext)

    acc = None
    for xi in range(6):
        term = (lax.slice_in_dim(ext, xi, xi + SLAB, axis=0) >> 6) + 9216 * xi
        acc = term if acc is None else jnp.minimum(acc, term)

    bits = cur & 63
    canon = bits >> 1
    label = jnp.where(acc < SENT, acc & 255, 0)
    cf = jnp.where(bits == 1, label, canon)

    j = (cf * 21846) >> 16
    r = cf - j * 3
    b0 = (j & 1) == 1
    b1 = (j & 2) == 2
    b2 = (j & 4) == 4
    m01 = jnp.where(b0, maps_ref[1, 0], maps_ref[0, 0])
    m23 = jnp.where(b0, maps_ref[3, 0], maps_ref[2, 0])
    m45 = jnp.where(b0, maps_ref[5, 0], maps_ref[4, 0])
    m67 = jnp.where(b0, maps_ref[7, 0], maps_ref[6, 0])
    m03 = jnp.where(b1, m23, m01)
    m47 = jnp.where(b1, m67, m45)
    w = jnp.where(b2, m47, m03)
    v = (w >> (r * 9)) & 511
    oi_ref[...] = v & 31
    os_ref[...] = v >> 5


def _final_call(maps, packed):
    bs = lambda im: pl.BlockSpec((SLAB, N, N), im)
    return pl.pallas_call(
        _final_body,
        grid=(NSLAB,),
        in_specs=[
            pl.BlockSpec(memory_space=pltpu.SMEM),
            bs(lambda s: (jnp.maximum(s - 1, 0), 0, 0)),
            bs(lambda s: (s, 0, 0)),
            bs(lambda s: (jnp.minimum(s + 1, NSLAB - 1), 0, 0)),
        ],
        out_specs=[bs(lambda s: (s, 0, 0)), bs(lambda s: (s, 0, 0))],
        out_shape=[jax.ShapeDtypeStruct((N, N, N), jnp.int32)] * 2,
    )(maps, packed, packed, packed)


def kernel(geometry, instances, semantics, instance_ids_2d):
    ids_pad = jnp.concatenate(
        [instance_ids_2d.astype(jnp.int32), jnp.full((4,), 30, jnp.int32)])
    packed = _stencil_call(ids_pad, geometry, instances, semantics)
    hist = _hist_call(instances.reshape(-1), semantics.reshape(-1))
    maps = _maps_call(ids_pad, hist)
    out_inst, out_sem = _final_call(maps, packed)
    return out_inst, out_sem
